# baseline (speedup 1.0000x reference)
import functools

import jax
import jax.numpy as jnp
from jax import lax
from jax.experimental import pallas as pl
from jax.experimental.pallas import tpu as pltpu
from jax.experimental.pallas import tpu_sc as plsc

VOCAB = 1000000
DIM = 64
B_POS = 16384
B_NEG = 81920
B_TOT = B_POS + B_NEG

NC = 2
NS = 16
NW = NC * NS
PER_W = B_TOT // NW
CH = 192
NCH = PER_W // CH
HALF = CH // 2
GROUPS = CH // 16


def _sc_dot_kernel(idxv_hbm, idxu_hbm, subv_hbm, subu_hbm, tab, out_hbm,
                   idxv_all, idxu_all, subv_all, subu_all,
                   va, ua, vb, ub, zbuf,
                   semva, semua, semvb, semub):
    wid = lax.axis_index("s") * NC + lax.axis_index("c")
    pltpu.sync_copy(idxv_hbm.at[wid], idxv_all)
    pltpu.sync_copy(idxu_hbm.at[wid], idxu_all)
    pltpu.sync_copy(subv_hbm.at[wid], subv_all)
    pltpu.sync_copy(subu_hbm.at[wid], subu_all)

    bufs = ((va, ua, semva, semua), (vb, ub, semvb, semub))

    def issue(t, bufset):
        vB, uB, sv, su = bufset
        for h in range(2):
            pltpu.async_copy(
                tab.at[idxv_all.at[t, h]], vB.at[pl.ds(h * HALF, HALF)], sv)
            pltpu.async_copy(
                tab.at[idxu_all.at[t, h]], uB.at[pl.ds(h * HALF, HALF)], su)

    def drain(bufset):
        vB, uB, sv, su = bufset
        for h in range(2):
            pltpu.make_async_copy(
                tab.at[pl.ds(0, HALF)], vB.at[pl.ds(h * HALF, HALF)], sv
            ).wait()
            pltpu.make_async_copy(
                tab.at[pl.ds(0, HALF)], uB.at[pl.ds(h * HALF, HALF)], su
            ).wait()

    def dot16(vB, uB, r, bv, bu):
        prod = jnp.zeros((16,), jnp.float32)
        for k in range(2):
            wv = vB[r, pl.ds(bv * 64 + k * 16, 16)]
            wu = uB[r, pl.ds(bu * 64 + 32 + k * 16, 16)]
            ve, vo = plsc.unpack(plsc.bitcast(wv, jnp.bfloat16),
                                 format=plsc.PackFormat.INTERLEAVED)
            ue, uo = plsc.unpack(plsc.bitcast(wu, jnp.bfloat16),
                                 format=plsc.PackFormat.INTERLEAVED)
            prod = prod + ve * ue + vo * uo
        return jnp.sum(prod)

    def compute(t, bufset):
        vB, uB = bufset[0], bufset[1]

        def gbody(g, carry):
            lane = lax.broadcasted_iota(jnp.int32, (16,), 0)
            base = pl.multiple_of(t * CH + g * 16, 16)
            svv = subv_all[pl.ds(base, 16)]
            suv = subu_all[pl.ds(base, 16)]
            acc = jnp.zeros((16,), jnp.float32)
            for j in range(16):
                r = g * 16 + j
                s = dot16(vB, uB, r, svv[j], suv[j])
                acc = jnp.where(lane == j, s, acc)
            zbuf[pl.ds(pl.multiple_of(g * 16, 16), 16)] = acc
            return carry

        lax.fori_loop(0, GROUPS, gbody, jnp.int32(0))
        pltpu.sync_copy(zbuf, out_hbm.at[wid, t])

    issue(0, bufs[0])
    issue(1, bufs[1])

    def chunk_pair(tp, carry):
        t0 = tp * 2
        drain(bufs[0])
        compute(t0, bufs[0])
        issue(jnp.minimum(t0 + 2, NCH - 1), bufs[0])
        drain(bufs[1])
        compute(t0 + 1, bufs[1])
        issue(jnp.minimum(t0 + 3, NCH - 1), bufs[1])
        return carry

    lax.fori_loop(0, NCH // 2, chunk_pair, jnp.int32(0))
    drain(bufs[0])
    drain(bufs[1])


def _sc_dot(idxv, idxu, subv, subu, tab2):
    mesh = plsc.VectorSubcoreMesh(core_axis_name="c", subcore_axis_name="s")
    k = functools.partial(
        pl.kernel,
        mesh=mesh,
        compiler_params=pltpu.CompilerParams(
            needs_layout_passes=False, use_tc_tiling_on_sc=True),
        out_type=jax.ShapeDtypeStruct((NW, NCH, CH), jnp.float32),
        scratch_types=[
            pltpu.VMEM((NCH, 2, HALF), jnp.int32),
            pltpu.VMEM((NCH, 2, HALF), jnp.int32),
            pltpu.VMEM((PER_W,), jnp.int32),
            pltpu.VMEM((PER_W,), jnp.int32),
            pltpu.VMEM((CH, 128), jnp.uint32),
            pltpu.VMEM((CH, 128), jnp.uint32),
            pltpu.VMEM((CH, 128), jnp.uint32),
            pltpu.VMEM((CH, 128), jnp.uint32),
            pltpu.VMEM((CH,), jnp.float32),
            pltpu.SemaphoreType.DMA,
            pltpu.SemaphoreType.DMA,
            pltpu.SemaphoreType.DMA,
            pltpu.SemaphoreType.DMA,
        ],
    )(_sc_dot_kernel)
    return k(idxv, idxu, subv, subu, tab2)


TRW = 512
NBLK = 977
SPLIT = TRW * NBLK


def _to_words(x):
    u = lax.bitcast_convert_type(x, jnp.uint32)
    rb = (u + jnp.uint32(0x7FFF) + ((u >> 16) & jnp.uint32(1))) >> 16
    return (rb[:, 0:DIM // 2] << 16) | rb[:, DIM // 2:DIM]


def _merge_body(vlo_ref, ulo_ref, vhi_ref, uhi_ref, o_ref):
    o_ref[:, 0:32] = _to_words(vlo_ref[...].T)
    o_ref[:, 32:64] = _to_words(ulo_ref[...].T)
    o_ref[:, 64:96] = _to_words(vhi_ref[...].T)
    o_ref[:, 96:128] = _to_words(uhi_ref[...].T)


def _merge_transpose(v_table, u_table):
    vtT = v_table.T
    utT = u_table.T
    return pl.pallas_call(
        _merge_body,
        grid=(NBLK,),
        in_specs=[
            pl.BlockSpec((DIM, TRW), lambda i: (0, i)),
            pl.BlockSpec((DIM, TRW), lambda i: (0, i)),
            pl.BlockSpec((DIM, TRW), lambda i: (0, i + NBLK)),
            pl.BlockSpec((DIM, TRW), lambda i: (0, i + NBLK)),
        ],
        out_specs=pl.BlockSpec((TRW, 2 * DIM), lambda i: (i, 0)),
        out_shape=jax.ShapeDtypeStruct((SPLIT, 2 * DIM), jnp.uint32),
        compiler_params=pltpu.CompilerParams(
            dimension_semantics=("parallel",)),
    )(vtT, utT, vtT, utT)


def _loss_body(z_ref, o_ref):
    z = z_ref[...]
    rows = lax.broadcasted_iota(jnp.int32, z.shape, 0)
    sign = jnp.where(rows < (B_POS // 128), 1.0, -1.0)
    x = sign * z
    a = jnp.minimum(x, 0.0) - jnp.log1p(jnp.exp(-jnp.abs(x)))
    o_ref[0, 0] = -jnp.sum(a)


def kernel(pos_v, pos_u, neg_v, neg_u, v_table, u_table):
    idx_v = jnp.concatenate([pos_v, neg_v]).astype(jnp.int32)
    idx_u = jnp.concatenate([pos_u, neg_u]).astype(jnp.int32)
    tab2 = _merge_transpose(v_table, u_table)
    hv = (idx_v >= SPLIT).astype(jnp.int32)
    hu = (idx_u >= SPLIT).astype(jnp.int32)
    idxv = (idx_v - hv * SPLIT).reshape(NW, NCH, 2, HALF)
    idxu = (idx_u - hu * SPLIT).reshape(NW, NCH, 2, HALF)
    subv = hv.reshape(NW, PER_W)
    subu = hu.reshape(NW, PER_W)
    z = _sc_dot(idxv, idxu, subv, subu, tab2)
    z2 = z.reshape(B_TOT // 128, 128)
    loss = pl.pallas_call(
        _loss_body,
        out_shape=jax.ShapeDtypeStruct((1, 1), jnp.float32),
        out_specs=pl.BlockSpec(memory_space=pltpu.SMEM),
    )(z2)
    return loss[0, 0]


# speedup vs baseline: 1.3415x; 1.3415x over previous
import functools

import jax
import jax.numpy as jnp
from jax import lax
from jax.experimental import pallas as pl
from jax.experimental.pallas import tpu as pltpu
from jax.experimental.pallas import tpu_sc as plsc

VOCAB = 1000000
DIM = 64
B_POS = 16384
B_NEG = 81920
B_TOT = B_POS + B_NEG

NC = 2
NS = 16
NW = NC * NS
PER_W = B_TOT // NW
CH = 192
NCH = PER_W // CH
HALF = CH // 2
GROUPS = CH // 16


def _sc_dot_kernel(idxv_hbm, idxu_hbm, subv_hbm, subu_hbm, tab, out_hbm,
                   idxv_all, idxu_all, subv_all, subu_all,
                   va, ua, vb, ub, zbuf,
                   semva, semua, semvb, semub):
    wid = lax.axis_index("s") * NC + lax.axis_index("c")
    pltpu.sync_copy(idxv_hbm.at[wid], idxv_all)
    pltpu.sync_copy(idxu_hbm.at[wid], idxu_all)
    pltpu.sync_copy(subv_hbm.at[wid], subv_all)
    pltpu.sync_copy(subu_hbm.at[wid], subu_all)

    bufs = ((va, ua, semva, semua), (vb, ub, semvb, semub))

    def issue(t, bufset):
        vB, uB, sv, su = bufset
        for h in range(2):
            pltpu.async_copy(
                tab.at[idxv_all.at[t, h]], vB.at[pl.ds(h * HALF, HALF)], sv)
            pltpu.async_copy(
                tab.at[idxu_all.at[t, h]], uB.at[pl.ds(h * HALF, HALF)], su)

    def drain(bufset):
        vB, uB, sv, su = bufset
        for h in range(2):
            pltpu.make_async_copy(
                tab.at[pl.ds(0, HALF)], vB.at[pl.ds(h * HALF, HALF)], sv
            ).wait()
            pltpu.make_async_copy(
                tab.at[pl.ds(0, HALF)], uB.at[pl.ds(h * HALF, HALF)], su
            ).wait()

    def dot16(vB, uB, r, bv, bu):
        prod = jnp.zeros((16,), jnp.float32)
        for k in range(2):
            wv = vB[r, pl.ds(bv * 64 + k * 16, 16)]
            wu = uB[r, pl.ds(bu * 64 + 32 + k * 16, 16)]
            ve, vo = plsc.unpack(plsc.bitcast(wv, jnp.bfloat16),
                                 format=plsc.PackFormat.INTERLEAVED)
            ue, uo = plsc.unpack(plsc.bitcast(wu, jnp.bfloat16),
                                 format=plsc.PackFormat.INTERLEAVED)
            prod = prod + ve * ue + vo * uo
        return jnp.sum(prod)

    def compute(t, bufset):
        vB, uB = bufset[0], bufset[1]

        def gbody(g, carry):
            lane = lax.broadcasted_iota(jnp.int32, (16,), 0)
            base = pl.multiple_of(t * CH + g * 16, 16)
            svv = subv_all[pl.ds(base, 16)]
            suv = subu_all[pl.ds(base, 16)]
            acc = jnp.zeros((16,), jnp.float32)
            for j in range(16):
                r = g * 16 + j
                s = dot16(vB, uB, r, svv[j], suv[j])
                acc = jnp.where(lane == j, s, acc)
            zbuf[pl.ds(pl.multiple_of(g * 16, 16), 16)] = acc
            return carry

        lax.fori_loop(0, GROUPS, gbody, jnp.int32(0))
        pltpu.sync_copy(zbuf, out_hbm.at[wid, t])

    issue(0, bufs[0])
    issue(1, bufs[1])

    def chunk_pair(tp, carry):
        t0 = tp * 2
        drain(bufs[0])
        compute(t0, bufs[0])
        issue(jnp.minimum(t0 + 2, NCH - 1), bufs[0])
        drain(bufs[1])
        compute(t0 + 1, bufs[1])
        issue(jnp.minimum(t0 + 3, NCH - 1), bufs[1])
        return carry

    lax.fori_loop(0, NCH // 2, chunk_pair, jnp.int32(0))
    drain(bufs[0])
    drain(bufs[1])


def _sc_dot(idxv, idxu, subv, subu, tab2):
    mesh = plsc.VectorSubcoreMesh(core_axis_name="c", subcore_axis_name="s")
    k = functools.partial(
        pl.kernel,
        mesh=mesh,
        compiler_params=pltpu.CompilerParams(
            needs_layout_passes=False, use_tc_tiling_on_sc=True),
        out_type=jax.ShapeDtypeStruct((NW, NCH, CH), jnp.float32),
        scratch_types=[
            pltpu.VMEM((NCH, 2, HALF), jnp.int32),
            pltpu.VMEM((NCH, 2, HALF), jnp.int32),
            pltpu.VMEM((PER_W,), jnp.int32),
            pltpu.VMEM((PER_W,), jnp.int32),
            pltpu.VMEM((CH, 128), jnp.uint32),
            pltpu.VMEM((CH, 128), jnp.uint32),
            pltpu.VMEM((CH, 128), jnp.uint32),
            pltpu.VMEM((CH, 128), jnp.uint32),
            pltpu.VMEM((CH,), jnp.float32),
            pltpu.SemaphoreType.DMA,
            pltpu.SemaphoreType.DMA,
            pltpu.SemaphoreType.DMA,
            pltpu.SemaphoreType.DMA,
        ],
    )(_sc_dot_kernel)
    return k(idxv, idxu, subv, subu, tab2)


TRW = 4096
NBLK = 123
SPLIT = TRW * NBLK


def _to_words(x):
    u = lax.bitcast_convert_type(x, jnp.uint32)
    rb = (u + jnp.uint32(0x7FFF) + ((u >> 16) & jnp.uint32(1))) >> 16
    return (rb[:, 0:DIM // 2] << 16) | rb[:, DIM // 2:DIM]


def _merge_body(vlo_ref, ulo_ref, vhi_ref, uhi_ref, o_ref):
    o_ref[:, 0:32] = _to_words(vlo_ref[...].T)
    o_ref[:, 32:64] = _to_words(ulo_ref[...].T)
    o_ref[:, 64:96] = _to_words(vhi_ref[...].T)
    o_ref[:, 96:128] = _to_words(uhi_ref[...].T)


def _merge_transpose(v_table, u_table):
    vtT = v_table.T
    utT = u_table.T
    return pl.pallas_call(
        _merge_body,
        grid=(NBLK,),
        in_specs=[
            pl.BlockSpec((DIM, TRW), lambda i: (0, i)),
            pl.BlockSpec((DIM, TRW), lambda i: (0, i)),
            pl.BlockSpec((DIM, TRW), lambda i: (0, jnp.minimum(i + NBLK, 2 * NBLK - 2))),
            pl.BlockSpec((DIM, TRW), lambda i: (0, jnp.minimum(i + NBLK, 2 * NBLK - 2))),
        ],
        out_specs=pl.BlockSpec((TRW, 2 * DIM), lambda i: (i, 0)),
        out_shape=jax.ShapeDtypeStruct((SPLIT, 2 * DIM), jnp.uint32),
        compiler_params=pltpu.CompilerParams(
            dimension_semantics=("parallel",)),
    )(vtT, utT, vtT, utT)


def _loss_body(z_ref, o_ref):
    z = z_ref[...]
    rows = lax.broadcasted_iota(jnp.int32, z.shape, 0)
    sign = jnp.where(rows < (B_POS // 128), 1.0, -1.0)
    x = sign * z
    a = jnp.minimum(x, 0.0) - jnp.log1p(jnp.exp(-jnp.abs(x)))
    o_ref[0, 0] = -jnp.sum(a)


def kernel(pos_v, pos_u, neg_v, neg_u, v_table, u_table):
    idx_v = jnp.concatenate([pos_v, neg_v]).astype(jnp.int32)
    idx_u = jnp.concatenate([pos_u, neg_u]).astype(jnp.int32)
    tab2 = _merge_transpose(v_table, u_table)
    hv = (idx_v >= SPLIT).astype(jnp.int32)
    hu = (idx_u >= SPLIT).astype(jnp.int32)
    idxv = (idx_v - hv * SPLIT).reshape(NW, NCH, 2, HALF)
    idxu = (idx_u - hu * SPLIT).reshape(NW, NCH, 2, HALF)
    subv = hv.reshape(NW, PER_W)
    subu = hu.reshape(NW, PER_W)
    z = _sc_dot(idxv, idxu, subv, subu, tab2)
    z2 = z.reshape(B_TOT // 128, 128)
    loss = pl.pallas_call(
        _loss_body,
        out_shape=jax.ShapeDtypeStruct((1, 1), jnp.float32),
        out_specs=pl.BlockSpec(memory_space=pltpu.SMEM),
    )(z2)
    return loss[0, 0]


# speedup vs baseline: 2.4979x; 1.8620x over previous
import functools

import jax
import jax.numpy as jnp
from jax import lax
from jax.experimental import pallas as pl
from jax.experimental.pallas import tpu as pltpu
from jax.experimental.pallas import tpu_sc as plsc

VOCAB = 1000000
DIM = 64
B_POS = 16384
B_NEG = 81920
B_TOT = B_POS + B_NEG

NC = 2
NS = 16
NW = NC * NS
PER_W = B_TOT // NW
CH = 192
NCH = PER_W // CH
HALF = CH // 2
GROUPS = CH // 16


def _sc_dot_kernel(idxv_hbm, idxu_hbm, subv_hbm, subu_hbm, tab, out_hbm,
                   idxv_all, idxu_all, subv_all, subu_all,
                   va, ua, vb, ub, zbuf,
                   semva, semua, semvb, semub):
    wid = lax.axis_index("s") * NC + lax.axis_index("c")
    pltpu.sync_copy(idxv_hbm.at[wid], idxv_all)
    pltpu.sync_copy(idxu_hbm.at[wid], idxu_all)
    pltpu.sync_copy(subv_hbm.at[wid], subv_all)
    pltpu.sync_copy(subu_hbm.at[wid], subu_all)

    bufs = ((va, ua, semva, semua), (vb, ub, semvb, semub))

    def issue(t, bufset):
        vB, uB, sv, su = bufset
        for h in range(2):
            pltpu.async_copy(
                tab.at[idxv_all.at[t, h]], vB.at[pl.ds(h * HALF, HALF)], sv)
            pltpu.async_copy(
                tab.at[idxu_all.at[t, h]], uB.at[pl.ds(h * HALF, HALF)], su)

    def drain(bufset):
        vB, uB, sv, su = bufset
        for h in range(2):
            pltpu.make_async_copy(
                tab.at[pl.ds(0, HALF)], vB.at[pl.ds(h * HALF, HALF)], sv
            ).wait()
            pltpu.make_async_copy(
                tab.at[pl.ds(0, HALF)], uB.at[pl.ds(h * HALF, HALF)], su
            ).wait()

    def dot16(vB, uB, r, bv, bu):
        himask = jnp.full((16,), 0xFFFF0000, jnp.uint32)
        prod = jnp.zeros((16,), jnp.float32)
        for k in range(4):
            wv = vB[r, pl.ds(bv * 64 + k * 16, 16)]
            wu = uB[r, pl.ds(bu * 64 + k * 16, 16)]
            vvec = plsc.bitcast(wv & himask, jnp.float32)
            uvec = plsc.bitcast(wu << 16, jnp.float32)
            prod = prod + vvec * uvec
        return jnp.sum(prod)

    def compute(t, bufset):
        vB, uB = bufset[0], bufset[1]

        def gbody(g, carry):
            lane = lax.broadcasted_iota(jnp.int32, (16,), 0)
            base = pl.multiple_of(t * CH + g * 16, 16)
            svv = subv_all[pl.ds(base, 16)]
            suv = subu_all[pl.ds(base, 16)]
            acc = jnp.zeros((16,), jnp.float32)
            for j in range(16):
                r = g * 16 + j
                s = dot16(vB, uB, r, svv[j], suv[j])
                acc = jnp.where(lane == j, s, acc)
            zbuf[pl.ds(pl.multiple_of(g * 16, 16), 16)] = acc
            return carry

        lax.fori_loop(0, GROUPS, gbody, jnp.int32(0))
        pltpu.sync_copy(zbuf, out_hbm.at[wid, t])

    issue(0, bufs[0])
    issue(1, bufs[1])

    def chunk_pair(tp, carry):
        t0 = tp * 2
        drain(bufs[0])
        compute(t0, bufs[0])
        issue(jnp.minimum(t0 + 2, NCH - 1), bufs[0])
        drain(bufs[1])
        compute(t0 + 1, bufs[1])
        issue(jnp.minimum(t0 + 3, NCH - 1), bufs[1])
        return carry

    lax.fori_loop(0, NCH // 2, chunk_pair, jnp.int32(0))
    drain(bufs[0])
    drain(bufs[1])


def _sc_dot(idxv, idxu, subv, subu, tab2):
    mesh = plsc.VectorSubcoreMesh(core_axis_name="c", subcore_axis_name="s")
    k = functools.partial(
        pl.kernel,
        mesh=mesh,
        compiler_params=pltpu.CompilerParams(
            needs_layout_passes=False, use_tc_tiling_on_sc=True),
        out_type=jax.ShapeDtypeStruct((NW, NCH, CH), jnp.float32),
        scratch_types=[
            pltpu.VMEM((NCH, 2, HALF), jnp.int32),
            pltpu.VMEM((NCH, 2, HALF), jnp.int32),
            pltpu.VMEM((PER_W,), jnp.int32),
            pltpu.VMEM((PER_W,), jnp.int32),
            pltpu.VMEM((CH, 128), jnp.uint32),
            pltpu.VMEM((CH, 128), jnp.uint32),
            pltpu.VMEM((CH, 128), jnp.uint32),
            pltpu.VMEM((CH, 128), jnp.uint32),
            pltpu.VMEM((CH,), jnp.float32),
            pltpu.SemaphoreType.DMA,
            pltpu.SemaphoreType.DMA,
            pltpu.SemaphoreType.DMA,
            pltpu.SemaphoreType.DMA,
        ],
    )(_sc_dot_kernel)
    return k(idxv, idxu, subv, subu, tab2)


TRW = 4096
NBLK = 123
SPLIT = TRW * NBLK


def _tp(x_ref):
    eye = jnp.eye(DIM, dtype=jnp.float32)
    return lax.dot_general(x_ref[...], eye, (((0,), (0,)), ((), ())),
                           preferred_element_type=jnp.float32)


def _pack2(tv, tu):
    uv = lax.bitcast_convert_type(tv, jnp.uint32)
    uu = lax.bitcast_convert_type(tu, jnp.uint32)
    return (uv & jnp.uint32(0xFFFF0000)) | (uu >> 16)


def _merge_body(vlo_ref, ulo_ref, vhi_ref, uhi_ref, o_ref):
    o_ref[:, 0:DIM] = _pack2(_tp(vlo_ref), _tp(ulo_ref))
    o_ref[:, DIM:2 * DIM] = _pack2(_tp(vhi_ref), _tp(uhi_ref))


def _merge_transpose(v_table, u_table):
    vtT = v_table.T
    utT = u_table.T
    return pl.pallas_call(
        _merge_body,
        grid=(NBLK,),
        in_specs=[
            pl.BlockSpec((DIM, TRW), lambda i: (0, i)),
            pl.BlockSpec((DIM, TRW), lambda i: (0, i)),
            pl.BlockSpec((DIM, TRW), lambda i: (0, jnp.minimum(i + NBLK, 2 * NBLK - 2))),
            pl.BlockSpec((DIM, TRW), lambda i: (0, jnp.minimum(i + NBLK, 2 * NBLK - 2))),
        ],
        out_specs=pl.BlockSpec((TRW, 2 * DIM), lambda i: (i, 0)),
        out_shape=jax.ShapeDtypeStruct((SPLIT, 2 * DIM), jnp.uint32),
        compiler_params=pltpu.CompilerParams(
            dimension_semantics=("parallel",)),
    )(vtT, utT, vtT, utT)


def _loss_body(z_ref, o_ref):
    z = z_ref[...]
    rows = lax.broadcasted_iota(jnp.int32, z.shape, 0)
    sign = jnp.where(rows < (B_POS // 128), 1.0, -1.0)
    x = sign * z
    a = jnp.minimum(x, 0.0) - jnp.log1p(jnp.exp(-jnp.abs(x)))
    o_ref[0, 0] = -jnp.sum(a)


def kernel(pos_v, pos_u, neg_v, neg_u, v_table, u_table):
    idx_v = jnp.concatenate([pos_v, neg_v]).astype(jnp.int32)
    idx_u = jnp.concatenate([pos_u, neg_u]).astype(jnp.int32)
    tab2 = _merge_transpose(v_table, u_table)
    hv = (idx_v >= SPLIT).astype(jnp.int32)
    hu = (idx_u >= SPLIT).astype(jnp.int32)
    idxv = (idx_v - hv * SPLIT).reshape(NW, NCH, 2, HALF)
    idxu = (idx_u - hu * SPLIT).reshape(NW, NCH, 2, HALF)
    subv = hv.reshape(NW, PER_W)
    subu = hu.reshape(NW, PER_W)
    z = _sc_dot(idxv, idxu, subv, subu, tab2)
    z2 = z.reshape(B_TOT // 128, 128)
    loss = pl.pallas_call(
        _loss_body,
        out_shape=jax.ShapeDtypeStruct((1, 1), jnp.float32),
        out_specs=pl.BlockSpec(memory_space=pltpu.SMEM),
    )(z2)
    return loss[0, 0]


# speedup vs baseline: 2.6066x; 1.0435x over previous
import functools

import jax
import jax.numpy as jnp
from jax import lax
from jax.experimental import pallas as pl
from jax.experimental.pallas import tpu as pltpu
from jax.experimental.pallas import tpu_sc as plsc

VOCAB = 1000000
DIM = 64
B_POS = 16384
B_NEG = 81920
B_TOT = B_POS + B_NEG

NC = 2
NS = 16
NW = NC * NS
PER_W = B_TOT // NW
CH = 192
NCH = PER_W // CH
HALF = CH // 2
GROUPS = CH // 16


def _sc_dot_kernel(idxv_hbm, idxu_hbm, subv_hbm, subu_hbm, tab, out_hbm,
                   idxv_all, idxu_all, subv_all, subu_all,
                   va, ua, vb, ub, zbuf,
                   semva, semua, semvb, semub):
    wid = lax.axis_index("s") * NC + lax.axis_index("c")
    pltpu.sync_copy(idxv_hbm.at[wid], idxv_all)
    pltpu.sync_copy(idxu_hbm.at[wid], idxu_all)
    pltpu.sync_copy(subv_hbm.at[wid], subv_all)
    pltpu.sync_copy(subu_hbm.at[wid], subu_all)

    bufs = ((va, ua, semva, semua), (vb, ub, semvb, semub))

    def issue(t, bufset):
        vB, uB, sv, su = bufset
        for h in range(2):
            pltpu.async_copy(
                tab.at[idxv_all.at[t, h]], vB.at[pl.ds(h * HALF, HALF)], sv)
            pltpu.async_copy(
                tab.at[idxu_all.at[t, h]], uB.at[pl.ds(h * HALF, HALF)], su)

    def drain(bufset):
        vB, uB, sv, su = bufset
        for h in range(2):
            pltpu.make_async_copy(
                tab.at[pl.ds(0, HALF)], vB.at[pl.ds(h * HALF, HALF)], sv
            ).wait()
            pltpu.make_async_copy(
                tab.at[pl.ds(0, HALF)], uB.at[pl.ds(h * HALF, HALF)], su
            ).wait()

    def dot16(vB, uB, r, bv, bu):
        himask = jnp.full((16,), 0xFFFF0000, jnp.uint32)
        prod = jnp.zeros((16,), jnp.float32)
        for k in range(4):
            wv = vB[r, pl.ds(bv * 64 + k * 16, 16)]
            wu = uB[r, pl.ds(bu * 64 + k * 16, 16)]
            vvec = plsc.bitcast(wv & himask, jnp.float32)
            uvec = plsc.bitcast(wu << 16, jnp.float32)
            prod = prod + vvec * uvec
        return jnp.sum(prod)

    def compute(t, bufset):
        vB, uB = bufset[0], bufset[1]

        def gbody(g, carry):
            lane = lax.broadcasted_iota(jnp.int32, (16,), 0)
            base = pl.multiple_of(t * CH + g * 16, 16)
            svv = subv_all[pl.ds(base, 16)]
            suv = subu_all[pl.ds(base, 16)]
            acc = jnp.zeros((16,), jnp.float32)
            for j in range(16):
                r = g * 16 + j
                s = dot16(vB, uB, r, svv[j], suv[j])
                acc = jnp.where(lane == j, s, acc)
            zbuf[pl.ds(pl.multiple_of(g * 16, 16), 16)] = acc
            return carry

        lax.fori_loop(0, GROUPS, gbody, jnp.int32(0))
        pltpu.sync_copy(zbuf, out_hbm.at[wid, t])

    issue(0, bufs[0])
    issue(1, bufs[1])

    def chunk_pair(tp, carry):
        t0 = tp * 2
        drain(bufs[0])
        compute(t0, bufs[0])
        issue(jnp.minimum(t0 + 2, NCH - 1), bufs[0])
        drain(bufs[1])
        compute(t0 + 1, bufs[1])
        issue(jnp.minimum(t0 + 3, NCH - 1), bufs[1])
        return carry

    lax.fori_loop(0, NCH // 2, chunk_pair, jnp.int32(0))
    drain(bufs[0])
    drain(bufs[1])


def _sc_dot(idxv, idxu, subv, subu, tab2):
    mesh = plsc.VectorSubcoreMesh(core_axis_name="c", subcore_axis_name="s")
    k = functools.partial(
        pl.kernel,
        mesh=mesh,
        compiler_params=pltpu.CompilerParams(
            needs_layout_passes=False, use_tc_tiling_on_sc=True),
        out_type=jax.ShapeDtypeStruct((NW, NCH, CH), jnp.float32),
        scratch_types=[
            pltpu.VMEM((NCH, 2, HALF), jnp.int32),
            pltpu.VMEM((NCH, 2, HALF), jnp.int32),
            pltpu.VMEM((PER_W,), jnp.int32),
            pltpu.VMEM((PER_W,), jnp.int32),
            pltpu.VMEM((CH, 128), jnp.uint32),
            pltpu.VMEM((CH, 128), jnp.uint32),
            pltpu.VMEM((CH, 128), jnp.uint32),
            pltpu.VMEM((CH, 128), jnp.uint32),
            pltpu.VMEM((CH,), jnp.float32),
            pltpu.SemaphoreType.DMA,
            pltpu.SemaphoreType.DMA,
            pltpu.SemaphoreType.DMA,
            pltpu.SemaphoreType.DMA,
        ],
    )(_sc_dot_kernel)
    return k(idxv, idxu, subv, subu, tab2)


TRW = 8192
NBLK = 62
SPLIT = TRW * NBLK


def _tp(x_ref):
    eye = jnp.eye(DIM, dtype=jnp.float32)
    return lax.dot_general(x_ref[...], eye, (((0,), (0,)), ((), ())),
                           preferred_element_type=jnp.float32)


def _pack2(tv, tu):
    uv = lax.bitcast_convert_type(tv, jnp.uint32)
    uu = lax.bitcast_convert_type(tu, jnp.uint32)
    return (uv & jnp.uint32(0xFFFF0000)) | (uu >> 16)


def _merge_body(vlo_ref, ulo_ref, vhi_ref, uhi_ref, o_ref):
    o_ref[:, 0:DIM] = _pack2(_tp(vlo_ref), _tp(ulo_ref))
    o_ref[:, DIM:2 * DIM] = _pack2(_tp(vhi_ref), _tp(uhi_ref))


def _merge_transpose(v_table, u_table):
    vtT = v_table.T
    utT = u_table.T
    return pl.pallas_call(
        _merge_body,
        grid=(NBLK,),
        in_specs=[
            pl.BlockSpec((DIM, TRW), lambda i: (0, i)),
            pl.BlockSpec((DIM, TRW), lambda i: (0, i)),
            pl.BlockSpec((DIM, TRW), lambda i: (0, jnp.minimum(i + NBLK, 2 * NBLK - 2))),
            pl.BlockSpec((DIM, TRW), lambda i: (0, jnp.minimum(i + NBLK, 2 * NBLK - 2))),
        ],
        out_specs=pl.BlockSpec((TRW, 2 * DIM), lambda i: (i, 0)),
        out_shape=jax.ShapeDtypeStruct((SPLIT, 2 * DIM), jnp.uint32),
        compiler_params=pltpu.CompilerParams(
            dimension_semantics=("parallel",)),
    )(vtT, utT, vtT, utT)


def _loss_body(z_ref, o_ref):
    z = z_ref[...]
    rows = lax.broadcasted_iota(jnp.int32, z.shape, 0)
    sign = jnp.where(rows < (B_POS // 128), 1.0, -1.0)
    x = sign * z
    a = jnp.minimum(x, 0.0) - jnp.log1p(jnp.exp(-jnp.abs(x)))
    o_ref[0, 0] = -jnp.sum(a)


def kernel(pos_v, pos_u, neg_v, neg_u, v_table, u_table):
    idx_v = jnp.concatenate([pos_v, neg_v]).astype(jnp.int32)
    idx_u = jnp.concatenate([pos_u, neg_u]).astype(jnp.int32)
    tab2 = _merge_transpose(v_table, u_table)
    hv = (idx_v >= SPLIT).astype(jnp.int32)
    hu = (idx_u >= SPLIT).astype(jnp.int32)
    idxv = (idx_v - hv * SPLIT).reshape(NW, NCH, 2, HALF)
    idxu = (idx_u - hu * SPLIT).reshape(NW, NCH, 2, HALF)
    subv = hv.reshape(NW, PER_W)
    subu = hu.reshape(NW, PER_W)
    z = _sc_dot(idxv, idxu, subv, subu, tab2)
    z2 = z.reshape(B_TOT // 128, 128)
    loss = pl.pallas_call(
        _loss_body,
        out_shape=jax.ShapeDtypeStruct((1, 1), jnp.float32),
        out_specs=pl.BlockSpec(memory_space=pltpu.SMEM),
    )(z2)
    return loss[0, 0]


# speedup vs baseline: 2.6134x; 1.0026x over previous
import functools

import jax
import jax.numpy as jnp
from jax import lax
from jax.experimental import pallas as pl
from jax.experimental.pallas import tpu as pltpu
from jax.experimental.pallas import tpu_sc as plsc

VOCAB = 1000000
DIM = 64
B_POS = 16384
B_NEG = 81920
B_TOT = B_POS + B_NEG

NC = 2
NS = 16
NW = NC * NS
PER_W = B_TOT // NW
CH = 192
NCH = PER_W // CH
HALF = CH // 2
GROUPS = CH // 16


def _sc_dot_kernel(idxv_hbm, idxu_hbm, subv_hbm, subu_hbm, tab, out_hbm,
                   idxv_all, idxu_all, subv_all, subu_all,
                   va, ua, vb, ub, zbuf,
                   semva, semua, semvb, semub):
    wid = lax.axis_index("s") * NC + lax.axis_index("c")
    pltpu.sync_copy(idxv_hbm.at[wid], idxv_all)
    pltpu.sync_copy(idxu_hbm.at[wid], idxu_all)
    pltpu.sync_copy(subv_hbm.at[wid], subv_all)
    pltpu.sync_copy(subu_hbm.at[wid], subu_all)

    bufs = ((va, ua, semva, semua), (vb, ub, semvb, semub))

    def issue(t, bufset):
        vB, uB, sv, su = bufset
        for h in range(2):
            pltpu.async_copy(
                tab.at[idxv_all.at[t, h]], vB.at[pl.ds(h * HALF, HALF)], sv)
            pltpu.async_copy(
                tab.at[idxu_all.at[t, h]], uB.at[pl.ds(h * HALF, HALF)], su)

    def drain(bufset):
        vB, uB, sv, su = bufset
        for h in range(2):
            pltpu.make_async_copy(
                tab.at[pl.ds(0, HALF)], vB.at[pl.ds(h * HALF, HALF)], sv
            ).wait()
            pltpu.make_async_copy(
                tab.at[pl.ds(0, HALF)], uB.at[pl.ds(h * HALF, HALF)], su
            ).wait()

    def dot16(vB, uB, r, bv, bu):
        himask = jnp.full((16,), 0xFFFF0000, jnp.uint32)
        prod = jnp.zeros((16,), jnp.float32)
        for k in range(4):
            wv = vB[r, pl.ds(bv * 64 + k * 16, 16)]
            wu = uB[r, pl.ds(bu * 64 + k * 16, 16)]
            vvec = plsc.bitcast(wv & himask, jnp.float32)
            uvec = plsc.bitcast(wu << 16, jnp.float32)
            prod = prod + vvec * uvec
        return jnp.sum(prod)

    def compute(t, bufset):
        vB, uB = bufset[0], bufset[1]

        def gbody(g, carry):
            lane = lax.broadcasted_iota(jnp.int32, (16,), 0)
            base = pl.multiple_of(t * CH + g * 16, 16)
            svv = subv_all[pl.ds(base, 16)]
            suv = subu_all[pl.ds(base, 16)]
            acc = jnp.zeros((16,), jnp.float32)
            for j in range(16):
                r = g * 16 + j
                s = dot16(vB, uB, r, svv[j], suv[j])
                acc = jnp.where(lane == j, s, acc)
            zbuf[pl.ds(pl.multiple_of(g * 16, 16), 16)] = acc
            return carry

        lax.fori_loop(0, GROUPS, gbody, jnp.int32(0))
        pltpu.sync_copy(zbuf, out_hbm.at[wid, t])

    issue(0, bufs[0])
    issue(1, bufs[1])

    def chunk_pair(tp, carry):
        t0 = tp * 2
        drain(bufs[0])
        compute(t0, bufs[0])
        issue(jnp.minimum(t0 + 2, NCH - 1), bufs[0])
        drain(bufs[1])
        compute(t0 + 1, bufs[1])
        issue(jnp.minimum(t0 + 3, NCH - 1), bufs[1])
        return carry

    lax.fori_loop(0, NCH // 2, chunk_pair, jnp.int32(0))
    drain(bufs[0])
    drain(bufs[1])


def _sc_dot(idxv, idxu, subv, subu, tab2):
    mesh = plsc.VectorSubcoreMesh(core_axis_name="c", subcore_axis_name="s")
    k = functools.partial(
        pl.kernel,
        mesh=mesh,
        compiler_params=pltpu.CompilerParams(
            needs_layout_passes=False, use_tc_tiling_on_sc=True),
        out_type=jax.ShapeDtypeStruct((NW, NCH, CH), jnp.float32),
        scratch_types=[
            pltpu.VMEM((NCH, 2, HALF), jnp.int32),
            pltpu.VMEM((NCH, 2, HALF), jnp.int32),
            pltpu.VMEM((PER_W,), jnp.int32),
            pltpu.VMEM((PER_W,), jnp.int32),
            pltpu.VMEM((CH, 128), jnp.uint32),
            pltpu.VMEM((CH, 128), jnp.uint32),
            pltpu.VMEM((CH, 128), jnp.uint32),
            pltpu.VMEM((CH, 128), jnp.uint32),
            pltpu.VMEM((CH,), jnp.float32),
            pltpu.SemaphoreType.DMA,
            pltpu.SemaphoreType.DMA,
            pltpu.SemaphoreType.DMA,
            pltpu.SemaphoreType.DMA,
        ],
    )(_sc_dot_kernel)
    return k(idxv, idxu, subv, subu, tab2)


TRW = 16384
NBLK = 31
SPLIT = TRW * NBLK


def _tp(x_ref):
    eye = jnp.eye(DIM, dtype=jnp.float32)
    return lax.dot_general(x_ref[...], eye, (((0,), (0,)), ((), ())),
                           preferred_element_type=jnp.float32)


def _pack2(tv, tu):
    uv = lax.bitcast_convert_type(tv, jnp.uint32)
    uu = lax.bitcast_convert_type(tu, jnp.uint32)
    return (uv & jnp.uint32(0xFFFF0000)) | (uu >> 16)


def _merge_body(vlo_ref, ulo_ref, vhi_ref, uhi_ref, o_ref):
    o_ref[:, 0:DIM] = _pack2(_tp(vlo_ref), _tp(ulo_ref))
    o_ref[:, DIM:2 * DIM] = _pack2(_tp(vhi_ref), _tp(uhi_ref))


def _merge_transpose(v_table, u_table):
    vtT = v_table.T
    utT = u_table.T
    return pl.pallas_call(
        _merge_body,
        grid=(NBLK,),
        in_specs=[
            pl.BlockSpec((DIM, TRW), lambda i: (0, i)),
            pl.BlockSpec((DIM, TRW), lambda i: (0, i)),
            pl.BlockSpec((DIM, TRW), lambda i: (0, jnp.minimum(i + NBLK, 2 * NBLK - 2))),
            pl.BlockSpec((DIM, TRW), lambda i: (0, jnp.minimum(i + NBLK, 2 * NBLK - 2))),
        ],
        out_specs=pl.BlockSpec((TRW, 2 * DIM), lambda i: (i, 0)),
        out_shape=jax.ShapeDtypeStruct((SPLIT, 2 * DIM), jnp.uint32),
        compiler_params=pltpu.CompilerParams(
            dimension_semantics=("parallel",)),
    )(vtT, utT, vtT, utT)


def _loss_body(z_ref, o_ref):
    z = z_ref[...]
    rows = lax.broadcasted_iota(jnp.int32, z.shape, 0)
    sign = jnp.where(rows < (B_POS // 128), 1.0, -1.0)
    x = sign * z
    a = jnp.minimum(x, 0.0) - jnp.log1p(jnp.exp(-jnp.abs(x)))
    o_ref[0, 0] = -jnp.sum(a)


def kernel(pos_v, pos_u, neg_v, neg_u, v_table, u_table):
    idx_v = jnp.concatenate([pos_v, neg_v]).astype(jnp.int32)
    idx_u = jnp.concatenate([pos_u, neg_u]).astype(jnp.int32)
    tab2 = _merge_transpose(v_table, u_table)
    hv = (idx_v >= SPLIT).astype(jnp.int32)
    hu = (idx_u >= SPLIT).astype(jnp.int32)
    idxv = (idx_v - hv * SPLIT).reshape(NW, NCH, 2, HALF)
    idxu = (idx_u - hu * SPLIT).reshape(NW, NCH, 2, HALF)
    subv = hv.reshape(NW, PER_W)
    subu = hu.reshape(NW, PER_W)
    z = _sc_dot(idxv, idxu, subv, subu, tab2)
    z2 = z.reshape(B_TOT // 128, 128)
    loss = pl.pallas_call(
        _loss_body,
        out_shape=jax.ShapeDtypeStruct((1, 1), jnp.float32),
        out_specs=pl.BlockSpec(memory_space=pltpu.SMEM),
    )(z2)
    return loss[0, 0]


# speedup vs baseline: 2.6163x; 1.0011x over previous
import functools

import jax
import jax.numpy as jnp
from jax import lax
from jax.experimental import pallas as pl
from jax.experimental.pallas import tpu as pltpu
from jax.experimental.pallas import tpu_sc as plsc

VOCAB = 1000000
DIM = 64
B_POS = 16384
B_NEG = 81920
B_TOT = B_POS + B_NEG

NC = 2
NS = 16
NW = NC * NS
PER_W = B_TOT // NW
CH = 192
NCH = PER_W // CH
HALF = CH // 2
GROUPS = CH // 16


def _sc_dot_kernel(idxv_hbm, idxu_hbm, subv_hbm, subu_hbm, tab, out_hbm,
                   idxv_all, idxu_all, subv_all, subu_all,
                   va, ua, vb, ub, zbuf,
                   semva, semua, semvb, semub):
    wid = lax.axis_index("s") * NC + lax.axis_index("c")
    pltpu.sync_copy(idxv_hbm.at[wid], idxv_all)
    pltpu.sync_copy(idxu_hbm.at[wid], idxu_all)
    pltpu.sync_copy(subv_hbm.at[wid], subv_all)
    pltpu.sync_copy(subu_hbm.at[wid], subu_all)

    bufs = ((va, ua, semva, semua), (vb, ub, semvb, semub))

    def issue(t, bufset):
        vB, uB, sv, su = bufset
        for h in range(2):
            pltpu.async_copy(
                tab.at[idxv_all.at[t, h]], vB.at[pl.ds(h * HALF, HALF)], sv)
            pltpu.async_copy(
                tab.at[idxu_all.at[t, h]], uB.at[pl.ds(h * HALF, HALF)], su)

    def drain(bufset):
        vB, uB, sv, su = bufset
        for h in range(2):
            pltpu.make_async_copy(
                tab.at[pl.ds(0, HALF)], vB.at[pl.ds(h * HALF, HALF)], sv
            ).wait()
            pltpu.make_async_copy(
                tab.at[pl.ds(0, HALF)], uB.at[pl.ds(h * HALF, HALF)], su
            ).wait()

    def dot16(vB, uB, r, bv, bu):
        himask = jnp.full((16,), 0xFFFF0000, jnp.uint32)
        prod = jnp.zeros((16,), jnp.float32)
        for k in range(4):
            wv = vB[r, pl.ds(bv * 64 + k * 16, 16)]
            wu = uB[r, pl.ds(bu * 64 + k * 16, 16)]
            vvec = plsc.bitcast(wv & himask, jnp.float32)
            uvec = plsc.bitcast(wu << 16, jnp.float32)
            prod = prod + vvec * uvec
        return jnp.sum(prod)

    def compute(t, bufset):
        vB, uB = bufset[0], bufset[1]

        def gbody(g, carry):
            lane = lax.broadcasted_iota(jnp.int32, (16,), 0)
            base = pl.multiple_of(t * CH + g * 16, 16)
            svv = subv_all[pl.ds(base, 16)]
            suv = subu_all[pl.ds(base, 16)]
            acc = jnp.zeros((16,), jnp.float32)
            for j in range(16):
                r = g * 16 + j
                s = dot16(vB, uB, r, svv[j], suv[j])
                acc = jnp.where(lane == j, s, acc)
            zbuf[pl.ds(pl.multiple_of(g * 16, 16), 16)] = acc
            return carry

        lax.fori_loop(0, GROUPS, gbody, jnp.int32(0))
        pltpu.sync_copy(zbuf, out_hbm.at[wid, t])

    issue(0, bufs[0])
    issue(1, bufs[1])

    def chunk_pair(tp, carry):
        t0 = tp * 2
        drain(bufs[0])
        compute(t0, bufs[0])
        issue(jnp.minimum(t0 + 2, NCH - 1), bufs[0])
        drain(bufs[1])
        compute(t0 + 1, bufs[1])
        issue(jnp.minimum(t0 + 3, NCH - 1), bufs[1])
        return carry

    lax.fori_loop(0, NCH // 2, chunk_pair, jnp.int32(0))
    drain(bufs[0])
    drain(bufs[1])


def _sc_dot(idxv, idxu, subv, subu, tab2):
    mesh = plsc.VectorSubcoreMesh(core_axis_name="c", subcore_axis_name="s")
    k = functools.partial(
        pl.kernel,
        mesh=mesh,
        compiler_params=pltpu.CompilerParams(
            needs_layout_passes=False, use_tc_tiling_on_sc=True),
        out_type=jax.ShapeDtypeStruct((NW, NCH, CH), jnp.float32),
        scratch_types=[
            pltpu.VMEM((NCH, 2, HALF), jnp.int32),
            pltpu.VMEM((NCH, 2, HALF), jnp.int32),
            pltpu.VMEM((PER_W,), jnp.int32),
            pltpu.VMEM((PER_W,), jnp.int32),
            pltpu.VMEM((CH, 128), jnp.uint32),
            pltpu.VMEM((CH, 128), jnp.uint32),
            pltpu.VMEM((CH, 128), jnp.uint32),
            pltpu.VMEM((CH, 128), jnp.uint32),
            pltpu.VMEM((CH,), jnp.float32),
            pltpu.SemaphoreType.DMA,
            pltpu.SemaphoreType.DMA,
            pltpu.SemaphoreType.DMA,
            pltpu.SemaphoreType.DMA,
        ],
    )(_sc_dot_kernel)
    return k(idxv, idxu, subv, subu, tab2)


TRW = 16384
NBLK = 31
SPLIT = TRW * NBLK


def _tp(x_ref):
    eye = jnp.eye(DIM, dtype=jnp.float32)
    return lax.dot_general(x_ref[...], eye, (((0,), (0,)), ((), ())),
                           preferred_element_type=jnp.float32)


def _pack2(tv, tu):
    uv = lax.bitcast_convert_type(tv, jnp.uint32)
    uu = lax.bitcast_convert_type(tu, jnp.uint32)
    return (uv & jnp.uint32(0xFFFF0000)) | (uu >> 16)


def _merge_body(vlo_ref, ulo_ref, vhi_ref, uhi_ref, o_ref):
    o_ref[:, 0:DIM] = _pack2(_tp(vlo_ref), _tp(ulo_ref))
    o_ref[:, DIM:2 * DIM] = _pack2(_tp(vhi_ref), _tp(uhi_ref))


def _merge_transpose(v_table, u_table):
    vtT = v_table.T
    utT = u_table.T
    return pl.pallas_call(
        _merge_body,
        grid=(NBLK,),
        in_specs=[
            pl.BlockSpec((DIM, TRW), lambda i: (0, i)),
            pl.BlockSpec((DIM, TRW), lambda i: (0, i)),
            pl.BlockSpec((DIM, TRW),
                         lambda i: (0, jnp.minimum(i + NBLK, (VOCAB - 1) // TRW))),
            pl.BlockSpec((DIM, TRW),
                         lambda i: (0, jnp.minimum(i + NBLK, (VOCAB - 1) // TRW))),
        ],
        out_specs=pl.BlockSpec((TRW, 2 * DIM), lambda i: (i, 0)),
        out_shape=jax.ShapeDtypeStruct((SPLIT, 2 * DIM), jnp.uint32),
        compiler_params=pltpu.CompilerParams(
            dimension_semantics=("parallel",)),
    )(vtT, utT, vtT, utT)


def _loss_body(z_ref, o_ref):
    z = z_ref[...]
    rows = lax.broadcasted_iota(jnp.int32, z.shape, 0)
    sign = jnp.where(rows < (B_POS // 128), 1.0, -1.0)
    x = sign * z
    a = jnp.minimum(x, 0.0) - jnp.log1p(jnp.exp(-jnp.abs(x)))
    o_ref[0, 0] = -jnp.sum(a)


def kernel(pos_v, pos_u, neg_v, neg_u, v_table, u_table):
    idx_v = jnp.concatenate([pos_v, neg_v]).astype(jnp.int32)
    idx_u = jnp.concatenate([pos_u, neg_u]).astype(jnp.int32)
    tab2 = _merge_transpose(v_table, u_table)
    hv = (idx_v >= SPLIT).astype(jnp.int32)
    hu = (idx_u >= SPLIT).astype(jnp.int32)
    idxv = (idx_v - hv * SPLIT).reshape(NW, NCH, 2, HALF)
    idxu = (idx_u - hu * SPLIT).reshape(NW, NCH, 2, HALF)
    subv = hv.reshape(NW, PER_W)
    subu = hu.reshape(NW, PER_W)
    z = _sc_dot(idxv, idxu, subv, subu, tab2)
    z2 = z.reshape(B_TOT // 128, 128)
    loss = pl.pallas_call(
        _loss_body,
        out_shape=jax.ShapeDtypeStruct((1, 1), jnp.float32),
        out_specs=pl.BlockSpec(memory_space=pltpu.SMEM),
    )(z2)
    return loss[0, 0]
